# baseline (speedup 1.0000x reference)
import functools

import numpy as np
import jax
import jax.numpy as jnp
from jax.experimental import pallas as pl
from jax.experimental.pallas import tpu as pltpu

_NL = 5
_TAPS = 9


def _dft_consts(H, W):
    def cs(n):
        k = np.arange(n)
        ang = -2.0 * np.pi * np.outer(k, k) / n
        return np.cos(ang) / np.sqrt(n), np.sin(ang) / np.sqrt(n)

    hr, hi = cs(H)
    wr, wi = cs(W)
    fwd = np.block([[hr, -hi], [hi, hr]])
    inv = np.block([[hr, hi], [-hi, hr]])
    col = np.concatenate([wr, wi], axis=1)
    return (jnp.asarray(fwd, jnp.float32), jnp.asarray(inv, jnp.float32),
            jnp.asarray(col, jnp.float32))


def _premask_consts(H, W):
    i = np.arange(H * W) % W
    return jnp.asarray(np.stack([(i != W - 1), (i != 0)]).astype(np.float32),
                       jnp.bfloat16)


def _body(mhf_ref, mhi_ref, nw_ref, em_ref, w_ref, b_ref,
          mask_ref, gdc_ref, gloss_ref, gt_ref,
          out_ref, dcm_ref, lossm_ref,
          xg_ref, xm_ref, xp_ref, x_ref, k0_ref, dc_ref, *, H, W, G, F):
    L = H * W
    c = pl.program_id(1)
    mhf = mhf_ref[...]
    mhi = mhi_ref[...]
    nw = nw_ref[...]

    def fft2(z):
        a = jnp.dot(mhf, z, preferred_element_type=jnp.float32)
        p = jnp.dot(a, nw, preferred_element_type=jnp.float32)
        return jnp.concatenate(
            [p[0:H, 0:W] - p[H:, W:], p[0:H, W:] + p[H:, 0:W]], axis=0)

    def ifft2(z):
        a = jnp.dot(mhi, z, preferred_element_type=jnp.float32)
        p = jnp.dot(a, nw, preferred_element_type=jnp.float32)
        return jnp.concatenate(
            [p[0:H, 0:W] + p[H:, W:], p[H:, 0:W] - p[0:H, W:]], axis=0)

    @pl.when(c == 0)
    def _init():
        m = mask_ref[0, :, :]
        dc = gdc_ref[0, :, :] * m
        dc_ref[...] = dc
        dcm_ref[0, :, :] = dc
        lossm_ref[0, :, :] = gloss_ref[0, :, :] * m
        k0 = dc * fft2(gt_ref[0, :, :])
        k0_ref[...] = k0
        x_ref[...] = ifft2(k0)
        xg_ref[...] = jnp.zeros_like(xg_ref)
        xm_ref[...] = jnp.zeros_like(xm_ref)
        xp_ref[...] = jnp.zeros_like(xp_ref)

    ml = em_ref[0:1, :]
    mr = em_ref[1:2, :]

    xin = x_ref[...].reshape(2, L).astype(jnp.bfloat16)
    xg_ref[0:2, G:G + L] = xin
    xm_ref[0:2, G:G + L] = xin * ml
    xp_ref[0:2, G:G + L] = xin * mr

    h = None
    for l in range(_NL):
        acc = None
        for t in range(_TAPS):
            dkh = t // 3 - 1
            dkw = t % 3 - 1
            s = dkh * W + dkw
            src = xm_ref if dkw == -1 else (xp_ref if dkw == 1 else xg_ref)
            p = jnp.dot(w_ref[l * _TAPS + t], src[:, G + s:G + s + L],
                        preferred_element_type=jnp.float32)
            acc = p if acc is None else acc + p
        acc = acc + b_ref[l]
        if l < _NL - 1:
            a = jnp.maximum(acc, 0.0).astype(jnp.bfloat16)
            xg_ref[:, G:G + L] = a
            xm_ref[:, G:G + L] = a * ml
            xp_ref[:, G:G + L] = a * mr
        else:
            h = acc[0:2, :]

    x = x_ref[...] + h.reshape(2 * H, W)
    dc = dc_ref[...]
    y = ifft2((1.0 - dc) * fft2(x) + dc * k0_ref[...])
    x_ref[...] = y

    @pl.when(c == pl.num_programs(1) - 1)
    def _fin():
        out_ref[0, :, :] = y


def kernel(mask, gt, gdc_mask, gloss_mask, w_all, b_all):
    B, _, H, W = mask.shape
    L = H * W
    F = w_all.shape[1]
    G = ((W + 1 + 127) // 128) * 128
    nc = w_all.shape[0] // (_NL * _TAPS)

    mhf, mhi, nw = _dft_consts(H, W)
    em = _premask_consts(H, W)
    w_b16 = w_all.astype(jnp.bfloat16)

    mask_s = mask.reshape(B, 2 * H, W)
    gdc_s = gdc_mask.reshape(B, 2 * H, W)
    gloss_s = gloss_mask.reshape(B, 2 * H, W)
    gt_s = gt.reshape(B, 2 * H, W)

    img = pl.BlockSpec((1, 2 * H, W), lambda b, c: (b, 0, 0))

    def cst(shp):
        return pl.BlockSpec(shp, lambda b, c: (0,) * len(shp))

    osd = jax.ShapeDtypeStruct((B, 2 * H, W), jnp.float32)
    body = functools.partial(_body, H=H, W=W, G=G, F=F)

    out_s, dcm_s, lossm_s = pl.pallas_call(
        body,
        out_shape=(osd, osd, osd),
        grid=(B, nc),
        in_specs=[
            cst((2 * H, 2 * H)),
            cst((2 * H, 2 * H)),
            cst((W, 2 * W)),
            cst((2, L)),
            pl.BlockSpec((_NL * _TAPS, F, F), lambda b, c: (c, 0, 0)),
            pl.BlockSpec((_NL, F, 1), lambda b, c: (c, 0, 0)),
            img, img, img, img,
        ],
        out_specs=(img, img, img),
        scratch_shapes=[
            pltpu.VMEM((F, L + 2 * G), jnp.bfloat16),
            pltpu.VMEM((F, L + 2 * G), jnp.bfloat16),
            pltpu.VMEM((F, L + 2 * G), jnp.bfloat16),
            pltpu.VMEM((2 * H, W), jnp.float32),
            pltpu.VMEM((2 * H, W), jnp.float32),
            pltpu.VMEM((2 * H, W), jnp.float32),
        ],
        compiler_params=pltpu.CompilerParams(
            dimension_semantics=("parallel", "arbitrary"),
            vmem_limit_bytes=64 * 1024 * 1024),
    )(mhf, mhi, nw, em, w_b16, b_all, mask_s, gdc_s, gloss_s, gt_s)

    return (out_s.reshape(B, 2, H, W),
            lossm_s.reshape(B, 2, H, W),
            dcm_s.reshape(B, 2, H, W),
            mask)


# speedup vs baseline: 1.2967x; 1.2967x over previous
import functools

import numpy as np
import jax
import jax.numpy as jnp
from jax.experimental import pallas as pl
from jax.experimental.pallas import tpu as pltpu

_NL = 5
_TAPS = 9


def _dft_consts(H, W):
    def cs(n):
        k = np.arange(n)
        ang = -2.0 * np.pi * np.outer(k, k) / n
        return np.cos(ang) / np.sqrt(n), np.sin(ang) / np.sqrt(n)

    hr, hi = cs(H)
    wr, wi = cs(W)
    fwd = np.block([[hr, -hi], [hi, hr]])
    inv = np.block([[hr, hi], [-hi, hr]])
    col = np.concatenate([wr, wi], axis=1)
    return (jnp.asarray(fwd, jnp.float32), jnp.asarray(inv, jnp.float32),
            jnp.asarray(col, jnp.float32))


def _premask_consts(H, W):
    i = np.arange(H * W) % W
    return jnp.asarray(np.stack([(i != W - 1), (i != 0)]).astype(np.float32))


def _body(mhf_ref, mhi_ref, nw_ref, em_ref, w_ref, b_ref,
          mask_ref, gdc_ref, gloss_ref, gt_ref,
          out_ref, dcm_ref, lossm_ref,
          xg_ref, xm_ref, xp_ref, x_ref, k0_ref, dc_ref, *, H, W, G, F):
    L = H * W
    c = pl.program_id(1)
    mhf = mhf_ref[...]
    mhi = mhi_ref[...]
    nw = nw_ref[...]

    def fft2(z):
        a = jnp.dot(mhf, z, preferred_element_type=jnp.float32)
        p = jnp.dot(a, nw, preferred_element_type=jnp.float32)
        return jnp.concatenate(
            [p[0:H, 0:W] - p[H:, W:], p[0:H, W:] + p[H:, 0:W]], axis=0)

    def ifft2(z):
        a = jnp.dot(mhi, z, preferred_element_type=jnp.float32)
        p = jnp.dot(a, nw, preferred_element_type=jnp.float32)
        return jnp.concatenate(
            [p[0:H, 0:W] + p[H:, W:], p[H:, 0:W] - p[0:H, W:]], axis=0)

    @pl.when(c == 0)
    def _init():
        m = mask_ref[0, :, :]
        dc = gdc_ref[0, :, :] * m
        dc_ref[...] = dc
        dcm_ref[0, :, :] = dc
        lossm_ref[0, :, :] = gloss_ref[0, :, :] * m
        k0 = dc * fft2(gt_ref[0, :, :])
        k0_ref[...] = k0
        x_ref[...] = ifft2(k0)
        xg_ref[...] = jnp.zeros_like(xg_ref)
        xm_ref[...] = jnp.zeros_like(xm_ref)
        xp_ref[...] = jnp.zeros_like(xp_ref)

    ml = em_ref[0:1, :]
    mr = em_ref[1:2, :]

    xin = x_ref[...].reshape(2, L)
    xg_ref[0:2, G:G + L] = xin
    xm_ref[0:2, G:G + L] = xin * ml
    xp_ref[0:2, G:G + L] = xin * mr

    h = None
    for l in range(_NL):
        acc = None
        for t in range(_TAPS):
            dkh = t // 3 - 1
            dkw = t % 3 - 1
            s = dkh * W + dkw
            src = xm_ref if dkw == -1 else (xp_ref if dkw == 1 else xg_ref)
            p = jnp.dot(w_ref[l * _TAPS + t], src[:, G + s:G + s + L],
                        preferred_element_type=jnp.float32)
            acc = p if acc is None else acc + p
        acc = acc + b_ref[l]
        if l < _NL - 1:
            a = jnp.maximum(acc, 0.0)
            xg_ref[:, G:G + L] = a
            xm_ref[:, G:G + L] = a * ml
            xp_ref[:, G:G + L] = a * mr
        else:
            h = acc[0:2, :]

    x = x_ref[...] + h.reshape(2 * H, W)
    dc = dc_ref[...]
    y = ifft2((1.0 - dc) * fft2(x) + dc * k0_ref[...])
    x_ref[...] = y

    @pl.when(c == pl.num_programs(1) - 1)
    def _fin():
        out_ref[0, :, :] = y


def kernel(mask, gt, gdc_mask, gloss_mask, w_all, b_all):
    B, _, H, W = mask.shape
    L = H * W
    F = w_all.shape[1]
    G = ((W + 1 + 127) // 128) * 128
    nc = w_all.shape[0] // (_NL * _TAPS)

    mhf, mhi, nw = _dft_consts(H, W)
    em = _premask_consts(H, W)

    mask_s = mask.reshape(B, 2 * H, W)
    gdc_s = gdc_mask.reshape(B, 2 * H, W)
    gloss_s = gloss_mask.reshape(B, 2 * H, W)
    gt_s = gt.reshape(B, 2 * H, W)

    img = pl.BlockSpec((1, 2 * H, W), lambda b, c: (b, 0, 0))

    def cst(shp):
        return pl.BlockSpec(shp, lambda b, c: (0,) * len(shp))

    osd = jax.ShapeDtypeStruct((B, 2 * H, W), jnp.float32)
    body = functools.partial(_body, H=H, W=W, G=G, F=F)

    out_s, dcm_s, lossm_s = pl.pallas_call(
        body,
        out_shape=(osd, osd, osd),
        grid=(B, nc),
        in_specs=[
            cst((2 * H, 2 * H)),
            cst((2 * H, 2 * H)),
            cst((W, 2 * W)),
            cst((2, L)),
            pl.BlockSpec((_NL * _TAPS, F, F), lambda b, c: (c, 0, 0)),
            pl.BlockSpec((_NL, F, 1), lambda b, c: (c, 0, 0)),
            img, img, img, img,
        ],
        out_specs=(img, img, img),
        scratch_shapes=[
            pltpu.VMEM((F, L + 2 * G), jnp.float32),
            pltpu.VMEM((F, L + 2 * G), jnp.float32),
            pltpu.VMEM((F, L + 2 * G), jnp.float32),
            pltpu.VMEM((2 * H, W), jnp.float32),
            pltpu.VMEM((2 * H, W), jnp.float32),
            pltpu.VMEM((2 * H, W), jnp.float32),
        ],
        compiler_params=pltpu.CompilerParams(
            dimension_semantics=("parallel", "arbitrary"),
            vmem_limit_bytes=64 * 1024 * 1024),
    )(mhf, mhi, nw, em, w_all, b_all, mask_s, gdc_s, gloss_s, gt_s)

    return (out_s.reshape(B, 2, H, W),
            lossm_s.reshape(B, 2, H, W),
            dcm_s.reshape(B, 2, H, W),
            mask)
